# baseline (speedup 1.0000x reference)
import functools

import jax
import jax.numpy as jnp
from jax import lax
from jax.experimental import pallas as pl
from jax.experimental.pallas import tpu as pltpu
from jax.experimental.pallas import tpu_sc as plsc

B = 4096
E = 128
SD = 128
NEG = 5
NP = 3 + 2 * NEG

NC = 2
NS = 16
NW = NC * NS
BPW = B // NW


@functools.cache
def _build_sc_gather():
    mesh = plsc.VectorSubcoreMesh(core_axis_name="c", subcore_axis_name="s")

    @functools.partial(
        pl.kernel,
        mesh=mesh,
        out_type=jax.ShapeDtypeStruct((NP * B, E), jnp.float32),
        scratch_types=[
            pltpu.VMEM((7, BPW), jnp.int32),
            pltpu.VMEM((4, BPW, E), jnp.float32),
            pltpu.SemaphoreType.DMA,
            pltpu.SemaphoreType.DMA,
            pltpu.SemaphoreType.DMA,
            pltpu.SemaphoreType.DMA,
            pltpu.SemaphoreType.DMA,
            pltpu.SemaphoreType.DMA,
            pltpu.SemaphoreType.DMA,
            pltpu.SemaphoreType.DMA,
        ],
    )
    def _sc_gather(idx_all_h, WI_h, WO_h, SE_h, out_h,
                   idx_v, bufs, g0, g1, g2, g3, w0, w1, w2, w3):
        wid = lax.axis_index("s") * NC + lax.axis_index("c")
        base = wid * BPW

        pltpu.sync_copy(idx_all_h.at[wid], idx_v)

        tasks = [(0, WI_h, 0), (1, WO_h, 1), (1, SE_h, 2)]
        for n in range(NEG):
            tasks.append((2 + n, WO_h, 3 + n))
        for n in range(NEG):
            tasks.append((2 + n, SE_h, 3 + NEG + n))

        NB = 4
        gsems = (g0, g1, g2, g3)
        wsems = (w0, w1, w2, w3)
        gcopies = [None] * NB
        wcopies = [None] * NB
        NT = len(tasks)

        def startg(t):
            slot = t % NB
            j, tab_h, _ = tasks[t]
            gcopies[slot] = pltpu.async_copy(tab_h.at[idx_v.at[j]],
                                             bufs.at[slot], gsems[slot])

        for t in range(NB):
            startg(t)
        for t in range(NT):
            slot = t % NB
            gcopies[slot].wait()
            _, _, p = tasks[t]
            wcopies[slot] = pltpu.async_copy(
                bufs.at[slot], out_h.at[pl.ds(p * B + base, BPW)],
                wsems[slot])
            r = t - 2
            if r >= 0 and r + NB < NT:
                wcopies[r % NB].wait()
                startg(r + NB)
        for t in range(max(0, NT - NB - 2), NT):
            if wcopies[t % NB] is not None:
                wcopies[t % NB].wait()
                wcopies[t % NB] = None

    return _sc_gather


BC = 2048


def _log_sigmoid(z):
    return jnp.minimum(z, 0.0) - jnp.log1p(jnp.exp(-jnp.abs(z)))


def _tc_body(g_r, f1w_r, f2w_r, f1b_r, f2b_r, out_r):
    i = pl.program_id(0)
    vI = g_r[0]
    U = jnp.dot(vI, f2w_r[...], preferred_element_type=jnp.float32)
    U1 = U[:, :E]
    U2 = U[:, E:]
    f1w = f1w_r[...]
    A1 = jnp.dot(U1, f1w, preferred_element_type=jnp.float32)
    A2 = jnp.dot(U2, f1w, preferred_element_type=jnp.float32)
    f1b = f1b_r[...]
    f2b = f2b_r[...]
    cI = jnp.sum(vI * f2b, axis=1)
    c1 = jnp.sum(U1 * f1b, axis=1)
    c2 = jnp.sum(U2 * f1b, axis=1)
    pos = jnp.sum(U1 * g_r[1] + A2 * g_r[2], axis=1) + c2 + cI
    part = -jnp.sum(_log_sigmoid(pos)) / B
    cneg = c1 + cI
    for n in range(NEG):
        zn = jnp.sum(A1 * g_r[3 + n] + U2 * g_r[3 + NEG + n], axis=1) + cneg
        part = part - jnp.sum(_log_sigmoid(-zn))

    @pl.when(i == 0)
    def _init():
        out_r[0, 0] = part

    @pl.when(i > 0)
    def _acc():
        out_r[0, 0] = out_r[0, 0] + part


def _tc_compute(g, f1w, f2w, f1b, f2b):
    grid = (B // BC,)
    return pl.pallas_call(
        _tc_body,
        grid=grid,
        in_specs=[
            pl.BlockSpec((NP, BC, E), lambda i: (0, i, 0)),
            pl.BlockSpec((SD, SD), lambda i: (0, 0)),
            pl.BlockSpec((E, E + SD), lambda i: (0, 0)),
            pl.BlockSpec((1, SD), lambda i: (0, 0)),
            pl.BlockSpec((1, E), lambda i: (0, 0)),
        ],
        out_specs=pl.BlockSpec((1, 1), lambda i: (0, 0),
                               memory_space=pltpu.SMEM),
        out_shape=jax.ShapeDtypeStruct((1, 1), jnp.float32),
    )(g, f1w, f2w, f1b, f2b)


def kernel(x, y, neg, WI, WO, seq_embed, fc1_w, fc1_b, fc2_w, fc2_b):
    xi = x.astype(jnp.int32).reshape(NW, 1, BPW)
    yi = y.astype(jnp.int32).reshape(NW, 1, BPW)
    negr = neg.astype(jnp.int32).reshape(NW, BPW, NEG).transpose(0, 2, 1)
    idx_all = jnp.concatenate([xi, yi, negr], axis=1)
    g = _build_sc_gather()(idx_all, WI, WO, seq_embed)
    out = _tc_compute(g.reshape(NP, B, E), fc1_w, fc2_w,
                      fc1_b.reshape(1, SD), fc2_b.reshape(1, E))
    return out[0, 0]


# speedup vs baseline: 1.0280x; 1.0280x over previous
import functools

import jax
import jax.numpy as jnp
from jax import lax
from jax.experimental import pallas as pl
from jax.experimental.pallas import tpu as pltpu
from jax.experimental.pallas import tpu_sc as plsc

B = 4096
E = 128
SD = 128
NEG = 5
NP = 3 + 2 * NEG

NC = 2
NS = 16
NW = NC * NS
BPW = B // NW


@functools.cache
def _build_sc_gather():
    mesh = plsc.VectorSubcoreMesh(core_axis_name="c", subcore_axis_name="s")

    @functools.partial(
        pl.kernel,
        mesh=mesh,
        out_type=jax.ShapeDtypeStruct((NP * B, E), jnp.float32),
        scratch_types=[
            pltpu.VMEM((7, BPW), jnp.int32),
            pltpu.VMEM((4, BPW, E), jnp.float32),
            pltpu.SemaphoreType.DMA,
            pltpu.SemaphoreType.DMA,
            pltpu.SemaphoreType.DMA,
            pltpu.SemaphoreType.DMA,
            pltpu.SemaphoreType.DMA,
            pltpu.SemaphoreType.DMA,
            pltpu.SemaphoreType.DMA,
            pltpu.SemaphoreType.DMA,
        ],
    )
    def _sc_gather(idx_all_h, WI_h, WO_h, SE_h, out_h,
                   idx_v, bufs, g0, g1, g2, g3, w0, w1, w2, w3):
        wid = lax.axis_index("s") * NC + lax.axis_index("c")
        base = wid * BPW

        pltpu.sync_copy(idx_all_h.at[wid], idx_v)

        tasks = [(0, WI_h, 0), (1, WO_h, 1), (1, SE_h, 2)]
        for n in range(NEG):
            tasks.append((2 + n, WO_h, 3 + n))
        for n in range(NEG):
            tasks.append((2 + n, SE_h, 3 + NEG + n))

        NB = 4
        gsems = (g0, g1, g2, g3)
        wsems = (w0, w1, w2, w3)
        gcopies = [None] * NB
        wcopies = [None] * NB
        NT = len(tasks)

        def startg(t):
            slot = t % NB
            j, tab_h, _ = tasks[t]
            gcopies[slot] = pltpu.async_copy(tab_h.at[idx_v.at[j]],
                                             bufs.at[slot], gsems[slot])

        for t in range(NB):
            startg(t)
        for t in range(NT):
            slot = t % NB
            gcopies[slot].wait()
            _, _, p = tasks[t]
            wcopies[slot] = pltpu.async_copy(
                bufs.at[slot], out_h.at[pl.ds(p * B + base, BPW)],
                wsems[slot])
            r = t - 2
            if r >= 0 and r + NB < NT:
                wcopies[r % NB].wait()
                startg(r + NB)
        for t in range(max(0, NT - NB - 2), NT):
            if wcopies[t % NB] is not None:
                wcopies[t % NB].wait()
                wcopies[t % NB] = None

    return _sc_gather


BC = 1024


def _log_sigmoid(z):
    return jnp.minimum(z, 0.0) - jnp.log1p(jnp.exp(-jnp.abs(z)))


def _tc_body(g_r, f1w_r, f2w_r, f1b_r, f2b_r, out_r):
    i = pl.program_id(0)
    vI = g_r[0]
    U = jnp.dot(vI, f2w_r[...], preferred_element_type=jnp.float32)
    U1 = U[:, :E]
    U2 = U[:, E:]
    f1w = f1w_r[...]
    A1 = jnp.dot(U1, f1w, preferred_element_type=jnp.float32)
    A2 = jnp.dot(U2, f1w, preferred_element_type=jnp.float32)
    f1b = f1b_r[...]
    f2b = f2b_r[...]
    cI = jnp.sum(vI * f2b, axis=1)
    c1 = jnp.sum(U1 * f1b, axis=1)
    c2 = jnp.sum(U2 * f1b, axis=1)
    pos = jnp.sum(U1 * g_r[1] + A2 * g_r[2], axis=1) + c2 + cI
    part = -jnp.sum(_log_sigmoid(pos)) / B
    cneg = c1 + cI
    for n in range(NEG):
        zn = jnp.sum(A1 * g_r[3 + n] + U2 * g_r[3 + NEG + n], axis=1) + cneg
        part = part - jnp.sum(_log_sigmoid(-zn))

    @pl.when(i == 0)
    def _init():
        out_r[0, 0] = part

    @pl.when(i > 0)
    def _acc():
        out_r[0, 0] = out_r[0, 0] + part


def _tc_compute(g, f1w, f2w, f1b, f2b):
    grid = (B // BC,)
    return pl.pallas_call(
        _tc_body,
        grid=grid,
        in_specs=[
            pl.BlockSpec((NP, BC, E), lambda i: (0, i, 0)),
            pl.BlockSpec((SD, SD), lambda i: (0, 0)),
            pl.BlockSpec((E, E + SD), lambda i: (0, 0)),
            pl.BlockSpec((1, SD), lambda i: (0, 0)),
            pl.BlockSpec((1, E), lambda i: (0, 0)),
        ],
        out_specs=pl.BlockSpec((1, 1), lambda i: (0, 0),
                               memory_space=pltpu.SMEM),
        out_shape=jax.ShapeDtypeStruct((1, 1), jnp.float32),
    )(g, f1w, f2w, f1b, f2b)


def kernel(x, y, neg, WI, WO, seq_embed, fc1_w, fc1_b, fc2_w, fc2_b):
    xi = x.astype(jnp.int32).reshape(NW, 1, BPW)
    yi = y.astype(jnp.int32).reshape(NW, 1, BPW)
    negr = neg.astype(jnp.int32).reshape(NW, BPW, NEG).transpose(0, 2, 1)
    idx_all = jnp.concatenate([xi, yi, negr], axis=1)
    g = _build_sc_gather()(idx_all, WI, WO, seq_embed)
    out = _tc_compute(g.reshape(NP, B, E), fc1_w, fc2_w,
                      fc1_b.reshape(1, SD), fc2_b.reshape(1, E))
    return out[0, 0]
